# baseline (speedup 1.0000x reference)
import jax
import jax.numpy as jnp
from jax import lax
from jax.experimental import pallas as pl
from jax.experimental.pallas import tpu as pltpu
from jax.experimental.pallas import tpu_sc as plsc

_VOCAB = 1_000_000
_DIM = 64
_B = 4096
_L = 200

_INFO = plsc.get_sparse_core_info()
_NW = _INFO.num_cores * _INFO.num_subcores
_ROWS_W = _B // _NW
_L0 = 128
_L1 = _L - _L0


def _embed_kernel(table_hbm, idx_hbm, out_hbm, idx_v, rows_v, sems_g, sems_o):
    wid = lax.axis_index("s") * _INFO.num_cores + lax.axis_index("c")
    base = wid * _ROWS_W

    pltpu.sync_copy(idx_hbm.at[pl.ds(base, _ROWS_W)], idx_v)

    def fill(b, h):
        pltpu.async_copy(table_hbm.at[idx_v.at[b, pl.ds(0, _L0)]],
                         rows_v.at[h, pl.ds(0, _L0)], sems_g[h])
        pltpu.async_copy(table_hbm.at[idx_v.at[b, pl.ds(_L0, _L1)]],
                         rows_v.at[h, pl.ds(_L0, _L1)], sems_g[h])

    def wait_fill(h):
        pltpu.make_async_copy(table_hbm.at[pl.ds(0, _L0)],
                              rows_v.at[h, pl.ds(0, _L0)], sems_g[h]).wait()
        pltpu.make_async_copy(table_hbm.at[pl.ds(0, _L1)],
                              rows_v.at[h, pl.ds(_L0, _L1)], sems_g[h]).wait()

    def writeback(b, h):
        pltpu.async_copy(rows_v.at[h], out_hbm.at[base + b], sems_o[h])

    def wait_writeback(b, h):
        pltpu.make_async_copy(rows_v.at[h], out_hbm.at[base + b],
                              sems_o[h]).wait()

    fill(0, 0)
    fill(1, 1)

    def round_body(r, _):
        for h in range(2):
            b = r * 2 + h
            wait_fill(h)
            writeback(b, h)

            @pl.when(b + 2 < _ROWS_W)
            def _():
                wait_writeback(b, h)
                fill(b + 2, h)

        return 0

    lax.fori_loop(0, _ROWS_W // 2, round_body, 0)

    for h in range(2):
        wait_writeback(_ROWS_W - 2 + h, h)


@jax.jit
def _embed(table, idx):
    mesh = plsc.VectorSubcoreMesh(core_axis_name="c", subcore_axis_name="s")
    run = pl.kernel(
        _embed_kernel,
        out_type=jax.ShapeDtypeStruct((_B, _L, 128), jnp.float32),
        mesh=mesh,
        scratch_types=[
            pltpu.VMEM((_ROWS_W, _L), jnp.int32),
            pltpu.VMEM((2, _L, 128), jnp.float32),
            [pltpu.SemaphoreType.DMA] * 2,
            [pltpu.SemaphoreType.DMA] * 2,
        ],
        compiler_params=pltpu.CompilerParams(use_tc_tiling_on_sc=True),
    )
    return run(table, idx)


def kernel(sent_words, embed_weight):
    if sent_words.dtype != jnp.int32:
        sent_words = sent_words.astype(jnp.int32)
    table = jnp.reshape(embed_weight, (_VOCAB // 2, 128))
    pairs = _embed(table, sent_words // 2)
    odd = (sent_words & 1)[:, :, None].astype(jnp.bool_)
    return jnp.where(odd, pairs[:, :, _DIM:], pairs[:, :, :_DIM])


# speedup vs baseline: 1.1899x; 1.1899x over previous
import jax
import jax.numpy as jnp
from jax import lax
from jax.experimental import pallas as pl
from jax.experimental.pallas import tpu as pltpu
from jax.experimental.pallas import tpu_sc as plsc

_VOCAB = 1_000_000
_DIM = 64
_B = 4096
_L = 200

_INFO = plsc.get_sparse_core_info()
_NW = _INFO.num_cores * _INFO.num_subcores
_ROWS_W = _B // _NW
_L0 = 128
_L1 = _L - _L0


def _embed_kernel(table_hbm, idx_hbm, out_hbm, idx_v, rows_v, sems_g, sems_o):
    wid = lax.axis_index("s") * _INFO.num_cores + lax.axis_index("c")
    base = wid * _ROWS_W

    pltpu.sync_copy(idx_hbm.at[pl.ds(base, _ROWS_W)], idx_v)

    def fill(b, h):
        pltpu.async_copy(table_hbm.at[idx_v.at[b, pl.ds(0, _L0)]],
                         rows_v.at[h, pl.ds(0, _L0)], sems_g[h])
        pltpu.async_copy(table_hbm.at[idx_v.at[b, pl.ds(_L0, _L1)]],
                         rows_v.at[h, pl.ds(_L0, _L1)], sems_g[h])

    def wait_fill(h):
        pltpu.make_async_copy(table_hbm.at[pl.ds(0, _L0)],
                              rows_v.at[h, pl.ds(0, _L0)], sems_g[h]).wait()
        pltpu.make_async_copy(table_hbm.at[pl.ds(0, _L1)],
                              rows_v.at[h, pl.ds(_L0, _L1)], sems_g[h]).wait()

    def writeback(b, h):
        pltpu.async_copy(rows_v.at[h], out_hbm.at[base + b], sems_o[h])

    def wait_writeback(b, h):
        pltpu.make_async_copy(rows_v.at[h], out_hbm.at[base + b],
                              sems_o[h]).wait()

    fill(0, 0)
    fill(1, 1)

    def round_body(r, _):
        for h in range(2):
            b = r * 2 + h
            wait_fill(h)
            writeback(b, h)

            @pl.when(b + 2 < _ROWS_W)
            def _():
                wait_writeback(b, h)
                fill(b + 2, h)

        return 0

    lax.fori_loop(0, _ROWS_W // 2, round_body, 0)

    for h in range(2):
        wait_writeback(_ROWS_W - 2 + h, h)


@jax.jit
def _embed(table, idx):
    mesh = plsc.VectorSubcoreMesh(core_axis_name="c", subcore_axis_name="s")
    run = pl.kernel(
        _embed_kernel,
        out_type=jax.ShapeDtypeStruct((_B, _L, _DIM), jnp.float32),
        mesh=mesh,
        scratch_types=[
            pltpu.VMEM((_ROWS_W, _L), jnp.int32),
            pltpu.VMEM((2, _L, _DIM), jnp.float32),
            [pltpu.SemaphoreType.DMA] * 2,
            [pltpu.SemaphoreType.DMA] * 2,
        ],
        compiler_params=pltpu.CompilerParams(use_tc_tiling_on_sc=False),
    )
    return run(table, idx)


def kernel(sent_words, embed_weight):
    if sent_words.dtype != jnp.int32:
        sent_words = sent_words.astype(jnp.int32)
    return _embed(embed_weight, sent_words)
